# baseline (speedup 1.0000x reference)
import jax
import jax.numpy as jnp
from jax.experimental import pallas as pl

_N = 10000
_G = 64


def _gcn(x, W, b, src, dst, n):
    x = x @ W
    loop = jnp.arange(n, dtype=src.dtype)
    src2 = jnp.concatenate([src, loop])
    dst2 = jnp.concatenate([dst, loop])
    deg = jax.ops.segment_sum(jnp.ones(src2.shape, dtype=x.dtype), dst2, num_segments=n)
    dinv = jnp.where(deg > 0, 1.0 / jnp.sqrt(jnp.maximum(deg, 1e-12)), 0.0)
    norm = dinv[src2] * dinv[dst2]
    msg = x[src2] * norm[:, None]
    out = jax.ops.segment_sum(msg, dst2, num_segments=n)
    return out + b


def _head_kernel(pooled_ref, wp_ref, bp_ref, out_ref):
    out_ref[...] = pooled_ref[...] @ wp_ref[...] + bp_ref[...]


def kernel(x, edge_index, batch, W1, b1, W2, b2, W3, b3, Wp, bp):
    src, dst = edge_index[0], edge_index[1]
    h = jax.nn.relu(_gcn(x, W1, b1, src, dst, _N))
    h = jax.nn.relu(_gcn(h, W2, b2, src, dst, _N))
    h = _gcn(h, W3, b3, src, dst, _N)
    sums = jax.ops.segment_sum(h, batch, num_segments=_G)
    cnt = jax.ops.segment_sum(jnp.ones((_N,), h.dtype), batch, num_segments=_G)
    pooled = sums / jnp.maximum(cnt, 1.0)[:, None]
    return pl.pallas_call(
        _head_kernel,
        out_shape=jax.ShapeDtypeStruct((_G, 1), jnp.float32),
    )(pooled, Wp, bp[None, :])


# speedup vs baseline: 7.7373x; 7.7373x over previous
import functools

import jax
import jax.numpy as jnp
from jax.experimental import pallas as pl

_N = 10000
_E = 320000
_G = 64
_NP = 10240

_BM = 512
_BK = 2048
_BP = 512


def _prep_body(h_ref, w_ref, dinv_ref, gf_ref, gb_ref):
    g = jnp.dot(h_ref[...], w_ref[...], preferred_element_type=jnp.float32)
    g = g * dinv_ref[...]
    gf_ref[...] = g
    gb_ref[...] = g.astype(jnp.bfloat16)


def _prep(h, w, dinv):
    n, d = h.shape
    return pl.pallas_call(
        _prep_body,
        grid=(n // _BP,),
        in_specs=[
            pl.BlockSpec((_BP, d), lambda i: (i, 0)),
            pl.BlockSpec((d, 256), lambda i: (0, 0)),
            pl.BlockSpec((_BP, 1), lambda i: (i, 0)),
        ],
        out_specs=[
            pl.BlockSpec((_BP, 256), lambda i: (i, 0)),
            pl.BlockSpec((_BP, 256), lambda i: (i, 0)),
        ],
        out_shape=[
            jax.ShapeDtypeStruct((n, 256), jnp.float32),
            jax.ShapeDtypeStruct((n, 256), jnp.bfloat16),
        ],
    )(h, w, dinv)


def _layer_body(m_ref, gk_ref, gi_ref, dinv_ref, b_ref, out_ref, *, relu, nk):
    k = pl.program_id(1)

    @pl.when(k == 0)
    def _init():
        out_ref[...] = jnp.zeros_like(out_ref)

    out_ref[...] += jnp.dot(m_ref[...], gk_ref[...],
                            preferred_element_type=jnp.float32)

    @pl.when(k == nk - 1)
    def _epilogue():
        acc = out_ref[...] + gi_ref[...]
        acc = acc * dinv_ref[...] + b_ref[...]
        if relu:
            acc = jnp.maximum(acc, 0.0)
        out_ref[...] = acc


def _layer(m_bf16, gb, gf, dinv, b, relu):
    nk = _NP // _BK
    body = functools.partial(_layer_body, relu=relu, nk=nk)
    return pl.pallas_call(
        body,
        grid=(_NP // _BM, nk),
        in_specs=[
            pl.BlockSpec((_BM, _BK), lambda i, k: (i, k)),
            pl.BlockSpec((_BK, 256), lambda i, k: (k, 0)),
            pl.BlockSpec((_BM, 256), lambda i, k: (i, 0)),
            pl.BlockSpec((_BM, 1), lambda i, k: (i, 0)),
            pl.BlockSpec((1, 256), lambda i, k: (0, 0)),
        ],
        out_specs=pl.BlockSpec((_BM, 256), lambda i, k: (i, 0)),
        out_shape=jax.ShapeDtypeStruct((_NP, 256), jnp.float32),
    )(m_bf16, gb, gf, dinv, b)


def _pool_body(h_ref, batch_ref, wp_ref, bp_ref, sums_ref, cnt_ref, out_ref,
               *, nblk):
    i = pl.program_id(0)

    @pl.when(i == 0)
    def _init():
        sums_ref[...] = jnp.zeros_like(sums_ref)
        cnt_ref[...] = jnp.zeros_like(cnt_ref)

    ids = jax.lax.broadcasted_iota(jnp.int32, (1, _G), 1)
    oh = (batch_ref[...] == ids).astype(jnp.float32)
    v = jnp.dot(h_ref[...], wp_ref[...], preferred_element_type=jnp.float32)
    sums_ref[...] += jax.lax.dot_general(
        oh, v, (((0,), (0,)), ((), ())), preferred_element_type=jnp.float32)
    cnt_ref[...] += jnp.sum(oh, axis=0)[:, None]

    @pl.when(i == nblk - 1)
    def _epilogue():
        out_ref[...] = (sums_ref[...] / jnp.maximum(cnt_ref[...], 1.0)
                        + bp_ref[...])


def _pool(h3, batch_pad, wp, bp):
    nblk = _NP // _BP
    body = functools.partial(_pool_body, nblk=nblk)
    sums, cnt, out = pl.pallas_call(
        body,
        grid=(nblk,),
        in_specs=[
            pl.BlockSpec((_BP, 256), lambda i: (i, 0)),
            pl.BlockSpec((_BP, 1), lambda i: (i, 0)),
            pl.BlockSpec((256, 1), lambda i: (0, 0)),
            pl.BlockSpec((1, 1), lambda i: (0, 0)),
        ],
        out_specs=[
            pl.BlockSpec((_G, 1), lambda i: (0, 0)),
            pl.BlockSpec((_G, 1), lambda i: (0, 0)),
            pl.BlockSpec((_G, 1), lambda i: (0, 0)),
        ],
        out_shape=[
            jax.ShapeDtypeStruct((_G, 1), jnp.float32),
            jax.ShapeDtypeStruct((_G, 1), jnp.float32),
            jax.ShapeDtypeStruct((_G, 1), jnp.float32),
        ],
    )(h3, batch_pad, wp, bp)
    return out


def kernel(x, edge_index, batch, W1, b1, W2, b2, W3, b3, Wp, bp):
    src, dst = edge_index[0], edge_index[1]

    m = jnp.zeros((_NP, _NP), jnp.float32).at[dst, src].add(1.0)
    m_bf16 = m.astype(jnp.bfloat16)
    deg = jax.ops.segment_sum(jnp.ones((_E,), jnp.float32), dst,
                              num_segments=_NP) + 1.0
    dinv = jax.lax.rsqrt(deg)[:, None]
    dinv = dinv.at[_N:].set(0.0)

    x_pad = jnp.zeros((_NP, 128), jnp.float32).at[:_N].set(x)

    gf, gb = _prep(x_pad, W1, dinv)
    h = _layer(m_bf16, gb, gf, dinv, b1[None, :], relu=True)
    gf, gb = _prep(h, W2, dinv)
    h = _layer(m_bf16, gb, gf, dinv, b2[None, :], relu=True)
    gf, gb = _prep(h, W3, dinv)
    h = _layer(m_bf16, gb, gf, dinv, b3[None, :], relu=False)

    batch_pad = jnp.full((_NP, 1), _G, jnp.int32).at[:_N, 0].set(batch)
    return _pool(h, batch_pad, Wp, bp[:, None])
